# baseline (speedup 1.0000x reference)
import functools

import jax
import jax.numpy as jnp
from jax import lax
from jax.experimental import pallas as pl
from jax.experimental.pallas import tpu as pltpu
from jax.experimental.pallas import tpu_sc as plsc

NUM_LINKS = 2000
NUM_PATHS = 7350
NUM_OD = 2450
BLK = 40
NBLK = NUM_LINKS // BLK
SUB = BLK // 8

NW = 32
OD_PER_W = 80
OD_PAD = NW * OD_PER_W
PATH_PAD = OD_PAD * 3
GROUPS = OD_PER_W // 16


def _pass1(link_params, D):
    def body(lp_ref, d_ref, t_ref, u_ref, uacc_ref):
        i = pl.program_id(0)
        lp = lp_ref[pl.ds(i * BLK, BLK), :]
        base = 1.0 + lp[:, 1:2] * (lp[:, 0:1] / lp[:, 4:5])
        t_blk = lp[:, 3:4] * jnp.exp(lp[:, 2:3] * jnp.log(base))
        t_ref[pl.ds(i * BLK, BLK), :] = t_blk

        @pl.when(i == 0)
        def _init():
            uacc_ref[...] = jnp.zeros_like(uacc_ref)

        u_loc = jnp.zeros((8, NUM_PATHS), jnp.float32)
        for k in range(SUB):
            u_loc = (u_loc +
                     d_ref[k * 8:(k + 1) * 8, :] * t_blk[k * 8:(k + 1) * 8, :])
        uacc_ref[...] += u_loc

        @pl.when(i == NBLK - 1)
        def _fin():
            u_ref[...] = jnp.sum(uacc_ref[...], axis=0, keepdims=True)

    return pl.pallas_call(
        body,
        grid=(NBLK,),
        in_specs=[
            pl.BlockSpec((NUM_LINKS, 8), lambda i: (0, 0)),
            pl.BlockSpec((BLK, NUM_PATHS), lambda i: (i, 0)),
        ],
        out_specs=[
            pl.BlockSpec((NUM_LINKS, 1), lambda i: (0, 0)),
            pl.BlockSpec((1, NUM_PATHS), lambda i: (0, 0)),
        ],
        out_shape=(
            jax.ShapeDtypeStruct((NUM_LINKS, 1), jnp.float32),
            jax.ShapeDtypeStruct((1, NUM_PATHS), jnp.float32),
        ),
        scratch_shapes=[pltpu.VMEM((8, NUM_PATHS), jnp.float32)],
        compiler_params=pltpu.CompilerParams(
            vmem_limit_bytes=128 * 1024 * 1024),
    )(link_params, D)


def _sc_softmax(u0_pad, u1_pad, u2_pad, q_pad):
    mesh = plsc.VectorSubcoreMesh(core_axis_name="c", subcore_axis_name="s")
    shp = jax.ShapeDtypeStruct((OD_PAD,), jnp.float32)
    vec = pltpu.VMEM((OD_PER_W,), jnp.float32)

    @functools.partial(
        pl.kernel, mesh=mesh,
        out_type=(shp,) * 6,
        scratch_types=[vec] * 10,
    )
    def k(u0_h, u1_h, u2_h, q_h, p0_h, p1_h, p2_h, f0_h, f1_h, f2_h,
          u0_v, u1_v, u2_v, q_v, p0_v, p1_v, p2_v, f0_v, f1_v, f2_v):
        wid = lax.axis_index("s") * 2 + lax.axis_index("c")
        base = wid * OD_PER_W
        sl_h = pl.ds(base, OD_PER_W)
        pltpu.sync_copy(u0_h.at[sl_h], u0_v)
        pltpu.sync_copy(u1_h.at[sl_h], u1_v)
        pltpu.sync_copy(u2_h.at[sl_h], u2_v)
        pltpu.sync_copy(q_h.at[sl_h], q_v)
        for g in range(GROUPS):
            sl = pl.ds(g * 16, 16)
            a = u0_v[sl]
            b = u1_v[sl]
            c = u2_v[sl]
            m = jnp.maximum(a, jnp.maximum(b, c))
            e0 = jnp.exp(a - m)
            e1 = jnp.exp(b - m)
            e2 = jnp.exp(c - m)
            r = 1.0 / (e0 + e1 + e2)
            p0, p1, p2 = e0 * r, e1 * r, e2 * r
            qg = q_v[sl]
            p0_v[sl] = p0
            p1_v[sl] = p1
            p2_v[sl] = p2
            f0_v[sl] = qg * p0
            f1_v[sl] = qg * p1
            f2_v[sl] = qg * p2
        pltpu.sync_copy(p0_v, p0_h.at[sl_h])
        pltpu.sync_copy(p1_v, p1_h.at[sl_h])
        pltpu.sync_copy(p2_v, p2_h.at[sl_h])
        pltpu.sync_copy(f0_v, f0_h.at[sl_h])
        pltpu.sync_copy(f1_v, f1_h.at[sl_h])
        pltpu.sync_copy(f2_v, f2_h.at[sl_h])

    return k(u0_pad, u1_pad, u2_pad, q_pad)


def _pass2(D, f_row):
    def body(f_ref, d_ref, x_ref):
        i = pl.program_id(0)
        f = f_ref[...]
        for k in range(SUB):
            d8 = d_ref[k * 8:(k + 1) * 8, :]
            x_ref[pl.ds(i * BLK + k * 8, 8), :] = jnp.sum(
                d8 * f, axis=1, keepdims=True)

    return pl.pallas_call(
        body,
        grid=(NBLK,),
        in_specs=[
            pl.BlockSpec((1, NUM_PATHS), lambda i: (0, 0)),
            pl.BlockSpec((BLK, NUM_PATHS), lambda i: (i, 0)),
        ],
        out_specs=pl.BlockSpec((NUM_LINKS, 1), lambda i: (0, 0)),
        out_shape=jax.ShapeDtypeStruct((NUM_LINKS, 1), jnp.float32),
        compiler_params=pltpu.CompilerParams(
            vmem_limit_bytes=128 * 1024 * 1024),
    )(f_row, D)


def kernel(x_hat, alpha, beta, q_hat, D, M, t_min, x_max):
    del M
    zeros = jnp.zeros((NUM_LINKS,), jnp.float32)
    link_params = jnp.stack(
        [x_hat, alpha, beta, t_min, x_max, zeros, zeros, zeros], axis=1)

    t2, u = _pass1(link_params, D)
    up = u.reshape(NUM_OD, 3)
    pad = lambda v: jnp.zeros((OD_PAD,), jnp.float32).at[:NUM_OD].set(v)
    p0, p1, p2, f0, f1, f2 = _sc_softmax(
        pad(up[:, 0]), pad(up[:, 1]), pad(up[:, 2]), pad(q_hat))
    p = jnp.stack([p0, p1, p2], axis=1).reshape(-1)[:NUM_PATHS]
    f = jnp.stack([f0, f1, f2], axis=1).reshape(-1)[:NUM_PATHS]
    x2 = _pass2(D, f.reshape(1, NUM_PATHS))
    return (x2.reshape(NUM_LINKS), t2.reshape(NUM_LINKS), f, p)


# speedup vs baseline: 2.3446x; 2.3446x over previous
import jax
import jax.numpy as jnp
from jax import lax
from jax.experimental import pallas as pl
from jax.experimental.pallas import tpu as pltpu

NUM_LINKS = 2000
NUM_PATHS = 7350
BLK = 80
NBLK = NUM_LINKS // BLK
SUB = BLK // 16


def _traffic_body(lp_ref, q3_ref, d_ref, xt_ref, f_ref, p_ref,
                  dscr_ref, uacc_ref):
    i = pl.program_id(0)

    lp = lp_ref[pl.ds(i * BLK, BLK), :]
    x_hat = lp[:, 0:1]
    alpha = lp[:, 1:2]
    beta = lp[:, 2:3]
    t_min = lp[:, 3:4]
    x_max = lp[:, 4:5]
    base = 1.0 + alpha * (x_hat / x_max)
    t_blk = t_min * jnp.exp(beta * jnp.log(base))
    xt_ref[pl.ds(i * BLK, BLK), 1:2] = t_blk

    @pl.when(i == 0)
    def _init():
        uacc_ref[...] = jnp.zeros_like(uacc_ref)

    u_loc = jnp.zeros((8, NUM_PATHS), jnp.float32)
    for k in range(SUB):
        d16 = d_ref[k * 16:(k + 1) * 16, :]
        u_loc = u_loc + d16[0:8, :] * t_blk[k * 16:k * 16 + 8, :]
        u_loc = u_loc + d16[8:16, :] * t_blk[k * 16 + 8:k * 16 + 16, :]
        dscr_ref[pl.ds(i * BLK + k * 16, 16), :] = d16.astype(jnp.bfloat16)
    uacc_ref[...] += u_loc

    @pl.when(i == NBLK - 1)
    def _finish():
        u = jnp.sum(uacc_ref[...], axis=0, keepdims=True)
        pos = lax.broadcasted_iota(jnp.int32, (1, NUM_PATHS), 1) % 3
        um1 = jnp.roll(u, 1, axis=1)
        um2 = jnp.roll(u, 2, axis=1)
        up1 = jnp.roll(u, -1, axis=1)
        up2 = jnp.roll(u, -2, axis=1)
        a = jnp.where(pos == 0, u, jnp.where(pos == 1, um1, um2))
        b = jnp.where(pos == 0, up1, jnp.where(pos == 1, u, um1))
        c = jnp.where(pos == 0, up2, jnp.where(pos == 1, up1, u))
        seg_max = jnp.maximum(a, jnp.maximum(b, c))
        e = jnp.exp(u - seg_max)
        em1 = jnp.roll(e, 1, axis=1)
        em2 = jnp.roll(e, 2, axis=1)
        ep1 = jnp.roll(e, -1, axis=1)
        ep2 = jnp.roll(e, -2, axis=1)
        denom = jnp.where(pos == 0, e + ep1 + ep2,
                          jnp.where(pos == 1, em1 + e + ep1, em2 + em1 + e))
        p = e / denom
        f = q3_ref[...] * p
        p_ref[...] = p
        f_ref[...] = f

        def p2_step(j, carry):
            d80 = dscr_ref[pl.ds(j * 80, 80), :].astype(jnp.float32)
            xt_ref[pl.ds(j * 80, 80), 0:1] = jnp.sum(
                d80 * f, axis=1, keepdims=True)
            return carry

        lax.fori_loop(0, NUM_LINKS // 80, p2_step, 0)


def kernel(x_hat, alpha, beta, q_hat, D, M, t_min, x_max):
    del M
    zeros = jnp.zeros((NUM_LINKS,), jnp.float32)
    link_params = jnp.stack(
        [x_hat, alpha, beta, t_min, x_max, zeros, zeros, zeros], axis=1)
    q3 = jnp.broadcast_to(q_hat[:, None], (q_hat.shape[0], 3))
    q3 = q3.reshape(1, NUM_PATHS)

    xt, f2, p2 = pl.pallas_call(
        _traffic_body,
        grid=(NBLK,),
        in_specs=[
            pl.BlockSpec((NUM_LINKS, 8), lambda i: (0, 0)),
            pl.BlockSpec((1, NUM_PATHS), lambda i: (0, 0)),
            pl.BlockSpec((BLK, NUM_PATHS), lambda i: (i, 0)),
        ],
        out_specs=[
            pl.BlockSpec((NUM_LINKS, 2), lambda i: (0, 0)),
            pl.BlockSpec((1, NUM_PATHS), lambda i: (0, 0)),
            pl.BlockSpec((1, NUM_PATHS), lambda i: (0, 0)),
        ],
        out_shape=(
            jax.ShapeDtypeStruct((NUM_LINKS, 2), jnp.float32),
            jax.ShapeDtypeStruct((1, NUM_PATHS), jnp.float32),
            jax.ShapeDtypeStruct((1, NUM_PATHS), jnp.float32),
        ),
        scratch_shapes=[
            pltpu.VMEM((NUM_LINKS, NUM_PATHS), jnp.bfloat16),
            pltpu.VMEM((8, NUM_PATHS), jnp.float32),
        ],
        compiler_params=pltpu.CompilerParams(
            vmem_limit_bytes=128 * 1024 * 1024),
    )(link_params, q3, D)

    return (xt[:, 0], xt[:, 1], f2.reshape(NUM_PATHS), p2.reshape(NUM_PATHS))
